# baseline (speedup 1.0000x reference)
import jax
import jax.numpy as jnp
from jax import lax
from jax.experimental import pallas as pl
from jax.experimental.pallas import tpu as pltpu
from jax.experimental.pallas import tpu_sc as plsc

D = 64
SEQ = 200
NB = 1024
W = 128

_info = plsc.get_sparse_core_info()
_NC, _NS, _L = _info.num_cores, _info.num_subcores, _info.num_lanes
NW = _NC * _NS
BPW = NB // NW
SPC = 2
CHUNK = SPC * BPW
NCHUNK = SEQ // SPC
NBUF = 4
NG = NCHUNK // NBUF
OPW = SEQ * D // W


def _sc_body(x_hbm, cos_hbm, sin_hbm, table_hbm, out_hbm,
             x_v, cos_v, sin_v, *ring):
    bufs = tuple(ring[7 * b:7 * b + 7] for b in range(NBUF))
    wid = lax.axis_index("s") * _NC + lax.axis_index("c")
    pltpu.sync_copy(x_hbm.at[wid], x_v)
    pltpu.sync_copy(cos_hbm, cos_v)
    pltpu.sync_copy(sin_hbm, sin_v)
    iota = lax.iota(jnp.int32, _L)
    perm = iota ^ 1
    obase = wid * BPW * OPW

    def build_gidx(c, b):
        idx_b, hv_b = bufs[b][0], bufs[b][1]
        for j in range(SPC):
            pos = c * SPC + j
            for h in range(BPW // _L):
                bvec = iota + (_L * h)
                fvec = bvec * SEQ + pos
                xv = plsc.load_gather(x_v, [fvec >> 7, fvec & 127])
                idx_b[pl.ds(j * BPW + _L * h, _L)] = xv >> 1
                hv_b[pl.ds(j * BPW + _L * h, _L)] = (xv & 1) * D

    def build_oidx(c, b):
        oidx_b = bufs[b][2]
        for h in range(BPW // _L):
            bvec = iota + (_L * h)
            oidx_b[pl.ds(_L * h, _L)] = obase + bvec * OPW + c

    def gather(c, b):
        pltpu.async_copy(table_hbm.at[bufs[b][0]], bufs[b][3], bufs[b][5])

    def gather_wait(c, b):
        pltpu.make_async_copy(table_hbm.at[bufs[b][0]], bufs[b][3],
                              bufs[b][5]).wait()

    def wb(c, b):
        pltpu.async_copy(bufs[b][4], out_hbm.at[bufs[b][2]], bufs[b][6])

    def wb_wait(c, b):
        pltpu.make_async_copy(bufs[b][4], out_hbm.at[bufs[b][2]],
                              bufs[b][6]).wait()

    for b in range(NBUF):
        build_gidx(b, b)
        gather(b, b)

    def group_body(g, carry):
        for b in range(NBUF):
            c = g * NBUF + b
            in_v, out_v = bufs[b][3], bufs[b][4]
            hv_b = bufs[b][1]
            gather_wait(c, b)

            @pl.when(g >= 1)
            def _():
                wb_wait(c - NBUF, b)

            build_oidx(c, b)

            for j in range(SPC):
                coff = j * D
                cc = [cos_v[c, pl.ds(coff + _L * k, _L)]
                      for k in range(D // _L)]
                ss = [sin_v[c, pl.ds(coff + _L * k, _L)]
                      for k in range(D // _L)]

                def row_body(i, carry2, j=j, cc=cc, ss=ss, in_v=in_v,
                             out_v=out_v, hv_b=hv_b):
                    r = j * BPW + i
                    rsplat = jnp.full((_L,), r, jnp.int32)
                    hsplat = plsc.load_gather(hv_b, [rsplat])
                    for k in range(D // _L):
                        col = hsplat + (_L * k) + iota
                        xv = plsc.load_gather(in_v, [rsplat, col])
                        sw = jnp.take_along_axis(xv, perm, axis=0,
                                                 mode="promise_in_bounds")
                        out_v[i, pl.ds(j * D + _L * k, _L)] = (
                            xv * cc[k] + sw * ss[k])
                    return carry2

                lax.fori_loop(0, BPW, row_body, 0, unroll=2)

            wb(c, b)

            @pl.when(c + NBUF < NCHUNK)
            def _():
                build_gidx(c + NBUF, b)
                gather(c + NBUF, b)
        return carry

    lax.fori_loop(0, NG, group_body, 0)
    for b in range(NBUF):
        wb_wait(NCHUNK - NBUF + b, b)


def kernel(x, table):
    nb, seq = x.shape
    x_w = x.astype(jnp.int32).reshape(NW, NCHUNK // 2, W)
    table_w = table.reshape(-1, W)

    inv_freq = 1.0 / (10000.0 ** (jnp.arange(0, D, 2, dtype=jnp.float32) / D))
    pos = jnp.arange(SEQ, dtype=jnp.float32)
    freqs = pos[:, None] * inv_freq[None, :]
    cos_t = jnp.repeat(jnp.cos(freqs), 2, axis=1).reshape(SEQ // 2, W)
    sign = jnp.tile(jnp.array([-1.0, 1.0], jnp.float32), D // 2)
    sin_t = (jnp.repeat(jnp.sin(freqs), 2, axis=1) * sign).reshape(SEQ // 2, W)

    ring_types = []
    for _ in range(NBUF):
        ring_types += [
            pltpu.VMEM((CHUNK,), jnp.int32),
            pltpu.VMEM((CHUNK,), jnp.int32),
            pltpu.VMEM((BPW,), jnp.int32),
            pltpu.VMEM((CHUNK, W), jnp.float32),
            pltpu.VMEM((BPW, W), jnp.float32),
            pltpu.SemaphoreType.DMA,
            pltpu.SemaphoreType.DMA,
        ]

    mesh = plsc.VectorSubcoreMesh(core_axis_name="c", subcore_axis_name="s")
    f = pl.kernel(
        _sc_body,
        out_type=jax.ShapeDtypeStruct((NB * OPW, W), jnp.float32),
        mesh=mesh,
        compiler_params=pltpu.CompilerParams(needs_layout_passes=False,
                                             skip_device_barrier=True),
        scratch_types=[
            pltpu.VMEM((NCHUNK // 2, W), jnp.int32),
            pltpu.VMEM((SEQ // 2, W), jnp.float32),
            pltpu.VMEM((SEQ // 2, W), jnp.float32),
        ] + ring_types,
    )
    out = f(x_w, cos_t, sin_t, table_w)
    return out.reshape(nb, seq, D)


# speedup vs baseline: 1.0569x; 1.0569x over previous
import jax
import jax.numpy as jnp
from jax import lax
from jax.experimental import pallas as pl
from jax.experimental.pallas import tpu as pltpu
from jax.experimental.pallas import tpu_sc as plsc

D = 64
SEQ = 200
NB = 1024
VOCAB = 1000000

_info = plsc.get_sparse_core_info()
_NC, _NS, _L = _info.num_cores, _info.num_subcores, _info.num_lanes
NW = _NC * _NS
BPW = NB // NW
SPC = 4
CHUNK = SPC * BPW
NCHUNK = SEQ // SPC
NBUF = 5
NIDX = 2 * NBUF
NG = NCHUNK // NBUF


def _sc_body(x_hbm, cos_hbm, sin_hbm, table_hbm, out_hbm,
             x_v, idx_v, oidx_v, cos_v, sin_v,
             in0, in1, in2, in3, in4, out0, out1, out2, out3, out4,
             si0, si1, si2, si3, si4, so0, so1, so2, so3, so4):
    wid = lax.axis_index("s") * _NC + lax.axis_index("c")
    base = wid * BPW * SEQ
    pltpu.sync_copy(x_hbm.at[pl.ds(wid * BPW, BPW), :], x_v)
    pltpu.sync_copy(cos_hbm, cos_v)
    pltpu.sync_copy(sin_hbm, sin_v)
    iota = lax.iota(jnp.int32, _L)
    perm = iota ^ 1

    bufs = ((in0, out0, si0, so0), (in1, out1, si1, so1),
            (in2, out2, si2, so2), (in3, out3, si3, so3),
            (in4, out4, si4, so4))

    def build_idx(c):
        slot = lax.rem(c, NIDX)
        for j in range(SPC):
            pos = c * SPC + j
            psplat = jnp.full((_L,), pos, jnp.int32)
            for h in range(BPW // _L):
                bvec = iota + (_L * h)
                vals = plsc.load_gather(x_v, [bvec, psplat])
                idx_v[slot, pl.ds(j * BPW + _L * h, _L)] = vals
                oidx_v[slot, pl.ds(j * BPW + _L * h, _L)] = (
                    base + bvec * SEQ + pos)

    def gather(c, b):
        pltpu.async_copy(table_hbm.at[idx_v.at[lax.rem(c, NIDX)]],
                         bufs[b][0], bufs[b][2])

    def gather_wait(c, b):
        pltpu.make_async_copy(table_hbm.at[idx_v.at[lax.rem(c, NIDX)]],
                              bufs[b][0], bufs[b][2]).wait()

    def wb(c, b):
        pltpu.async_copy(bufs[b][1], out_hbm.at[oidx_v.at[lax.rem(c, NIDX)]],
                         bufs[b][3])

    def wb_wait(c, b):
        pltpu.make_async_copy(bufs[b][1],
                              out_hbm.at[oidx_v.at[lax.rem(c, NIDX)]],
                              bufs[b][3]).wait()

    for b in range(NBUF):
        build_idx(b)
        gather(b, b)

    def group_body(g, carry):
        for b in range(NBUF):
            c = g * NBUF + b
            in_v, out_v = bufs[b][0], bufs[b][1]
            gather_wait(c, b)

            @pl.when(g >= 1)
            def _():
                wb_wait(c - NBUF, b)

            for j in range(SPC):
                s = c * SPC + j
                cc = [cos_v[s, pl.ds(_L * k, _L)] for k in range(D // _L)]
                ss = [sin_v[s, pl.ds(_L * k, _L)] for k in range(D // _L)]

                def row_body(i, carry2, j=j, cc=cc, ss=ss):
                    r = j * BPW + i
                    for k in range(D // _L):
                        xv = in_v[r, pl.ds(_L * k, _L)]
                        sw = jnp.take_along_axis(xv, perm, axis=0,
                                                 mode="promise_in_bounds")
                        out_v[r, pl.ds(_L * k, _L)] = xv * cc[k] + sw * ss[k]
                    return carry2

                lax.fori_loop(0, BPW, row_body, 0, unroll=2)

            wb(c, b)

            @pl.when(c + NBUF < NCHUNK)
            def _():
                build_idx(c + NBUF)
                gather(c + NBUF, b)
        return carry

    lax.fori_loop(0, NG, group_body, 0)
    for b in range(NBUF):
        wb_wait(NCHUNK - NBUF + b, b)


def kernel(x, table):
    nb, seq = x.shape
    x_i = x.astype(jnp.int32)

    inv_freq = 1.0 / (10000.0 ** (jnp.arange(0, D, 2, dtype=jnp.float32) / D))
    pos = jnp.arange(SEQ, dtype=jnp.float32)
    freqs = pos[:, None] * inv_freq[None, :]
    cos_t = jnp.repeat(jnp.cos(freqs), 2, axis=1)
    sign = jnp.tile(jnp.array([-1.0, 1.0], jnp.float32), D // 2)
    sin_t = jnp.repeat(jnp.sin(freqs), 2, axis=1) * sign

    mesh = plsc.VectorSubcoreMesh(core_axis_name="c", subcore_axis_name="s")
    f = pl.kernel(
        _sc_body,
        out_type=jax.ShapeDtypeStruct((NB * SEQ, D), jnp.float32),
        mesh=mesh,
        compiler_params=pltpu.CompilerParams(needs_layout_passes=False,
                                             use_tc_tiling_on_sc=False,
                                             skip_device_barrier=True),
        scratch_types=[
            pltpu.VMEM((BPW, SEQ), jnp.int32),
            pltpu.VMEM((NIDX, CHUNK), jnp.int32),
            pltpu.VMEM((NIDX, CHUNK), jnp.int32),
            pltpu.VMEM((SEQ, D), jnp.float32),
            pltpu.VMEM((SEQ, D), jnp.float32),
        ] + [pltpu.VMEM((CHUNK, D), jnp.float32)] * (2 * NBUF)
          + [pltpu.SemaphoreType.DMA] * (2 * NBUF),
    )
    out = f(x_i, cos_t, sin_t, table)
    return out.reshape(nb, seq, D)
